# baseline (speedup 1.0000x reference)
import functools

import jax
import jax.numpy as jnp
from jax import lax
from jax.experimental import pallas as pl
from jax.experimental.pallas import tpu as pltpu
from jax.experimental.pallas import tpu_sc as plsc

D = 1024
F = 256
E = 64
T = 2048
FS = 2048
BT = 128
G = 96
PAD = G * BT
NPAIR = 2 * T
TRASH = NPAIR

_NEG = -1e30


def _gate_body(x_ref, gw_ref, ei0_ref, ei1_ref, ew0_ref, ew1_ref):
    xb = x_ref[...]
    l = lax.dot_general(xb, gw_ref[...], (((1,), (1,)), ((), ())),
                        preferred_element_type=jnp.float32)
    m0 = jnp.max(l, axis=1, keepdims=True)
    z = jnp.sum(jnp.exp(l - m0), axis=1, keepdims=True)
    ids = lax.broadcasted_iota(jnp.int32, l.shape, 1)
    a0 = jnp.min(jnp.where(l == m0, ids, E), axis=1, keepdims=True)
    lm = jnp.where(ids == a0, _NEG, l)
    m1 = jnp.max(lm, axis=1, keepdims=True)
    a1 = jnp.min(jnp.where(lm == m1, ids, E), axis=1, keepdims=True)
    p0 = 1.0 / z
    p1 = jnp.exp(m1 - m0) / z
    e0 = jnp.exp(p0)
    e1 = jnp.exp(p1)
    s = e0 + e1
    ei0_ref[...] = a0
    ei1_ref[...] = a1
    ew0_ref[...] = e0 / s
    ew1_ref[...] = e1 / s


def _gate(x2d, gate_w):
    TB = 256
    grid = (T // TB,)
    out = pl.pallas_call(
        _gate_body,
        grid=grid,
        in_specs=[
            pl.BlockSpec((TB, D), lambda i: (i, 0)),
            pl.BlockSpec((E, D), lambda i: (0, 0)),
        ],
        out_specs=[
            pl.BlockSpec((TB, 1), lambda i: (i, 0)),
            pl.BlockSpec((TB, 1), lambda i: (i, 0)),
            pl.BlockSpec((TB, 1), lambda i: (i, 0)),
            pl.BlockSpec((TB, 1), lambda i: (i, 0)),
        ],
        out_shape=[
            jax.ShapeDtypeStruct((T, 1), jnp.int32),
            jax.ShapeDtypeStruct((T, 1), jnp.int32),
            jax.ShapeDtypeStruct((T, 1), jnp.float32),
            jax.ShapeDtypeStruct((T, 1), jnp.float32),
        ],
    )(x2d, gate_w)
    return out


_SC_MESH = plsc.VectorSubcoreMesh(core_axis_name="c", subcore_axis_name="s")
_EPW = 4
_RCH = 32
_WROWS = PAD // 32
_NCH = _WROWS // _RCH


def _route_body(ei_hbm, ew_hbm, pw_hbm, stok_hbm, gidx_hbm, texp_hbm,
                ei_v, ew_v, bins_v, pwbuf, stokbuf, gidxbuf, hist_v,
                texbuf):
    cid = lax.axis_index("c")
    sid = lax.axis_index("s")
    iota16 = lax.iota(jnp.int32, 16)

    @pl.when(cid == 0)
    def _():
        pltpu.sync_copy(ei_hbm, ei_v)
        pltpu.sync_copy(ew_hbm, ew_v)

        cnts = []
        for j4 in range(_EPW):
            e = sid * _EPW + j4

            def body(j, cnt, j4=j4, e=e):
                v = ei_v[pl.ds(pl.multiple_of(j * 16, 16), 16)]
                m = v == e
                mi = m.astype(jnp.int32)
                cs = plsc.cumsum(mi)
                pos = cnt + cs - mi + j4 * NPAIR
                plsc.store_scatter(bins_v, [pos], iota16 + j * 16, mask=m)
                return cnt + jnp.sum(mi)

            cnts.append(lax.fori_loop(0, NPAIR // 16, body, jnp.int32(0)))

        for r in range(4):
            hist_v[pl.ds(pl.multiple_of(16 * r, 16), 16)] = (
                jnp.zeros((16,), jnp.int32))
        ones16 = jnp.full((16,), 1, jnp.int32)

        def hbody(j, _):
            v = ei_v[pl.ds(pl.multiple_of(j * 16, 16), 16)]
            plsc.addupdate_scatter(hist_v, [v], ones16)
            return 0

        lax.fori_loop(0, NPAIR // 16, hbody, 0)

        t_vregs, ts_vregs = [], []
        base = jnp.int32(0)
        for r in range(4):
            c_r = hist_v[pl.ds(pl.multiple_of(16 * r, 16), 16)]
            t_r = lax.shift_right_logical(c_r + (BT - 1), 7)
            incl = plsc.cumsum(t_r)
            ts_r = incl - t_r + base
            base = base + jnp.sum(t_r)
            t_vregs.append(t_r)
            ts_vregs.append(ts_r)
        total_tiles = base

        for j4 in range(_EPW):
            e = sid * _EPW + j4
            cnt = cnts[j4]
            ts_e = jnp.int32(0)
            for r in range(4):
                ts_e = ts_e + jnp.sum(
                    jnp.where(iota16 + 16 * r == e, ts_vregs[r], 0))
            tcnt_e = lax.shift_right_logical(cnt + (BT - 1), 7)
            rowoff = ts_e * BT

            def fill(j, _, j4=j4, cnt=cnt):
                lanes = iota16 + j * 16
                m = lanes < cnt
                p = bins_v[pl.ds(j4 * NPAIR + j * 16, 16)]
                pv = jnp.where(m, p, 0)
                wv = jnp.where(m, plsc.load_gather(ew_v, [pv]), 0.0)
                off = pl.ds(pl.multiple_of(j * 16, 16), 16)
                pwbuf[off] = wv
                stokbuf[off] = jnp.where(m, pv, TRASH)
                gidxbuf[off] = lax.shift_right_logical(pv, 1)
                return 0

            lax.fori_loop(0, tcnt_e * (BT // 16), fill, 0)

            def dump(t2, _, rowoff=rowoff):
                src = pl.ds(pl.multiple_of(t2 * BT, BT), BT)
                dst = pl.ds(pl.multiple_of(rowoff + t2 * BT, BT), BT)
                pltpu.sync_copy(pwbuf.at[src], pw_hbm.at[dst])
                pltpu.sync_copy(stokbuf.at[src], stok_hbm.at[dst])
                pltpu.sync_copy(gidxbuf.at[src], gidx_hbm.at[dst])
                return 0

            lax.fori_loop(0, tcnt_e, dump, 0)

        for i in range(BT // 16):
            off = pl.ds(pl.multiple_of(i * 16, 16), 16)
            pwbuf[off] = jnp.zeros((16,), jnp.float32)
            stokbuf[off] = jnp.full((16,), TRASH, jnp.int32)
            gidxbuf[off] = jnp.zeros((16,), jnp.int32)
        ntail = jnp.maximum(0, (G - total_tiles - sid + 15)) // 16

        def tail(k, _):
            g = total_tiles + sid + 16 * k
            dst = pl.ds(pl.multiple_of(g * BT, BT), BT)
            src = pl.ds(0, BT)
            pltpu.sync_copy(pwbuf.at[src], pw_hbm.at[dst])
            pltpu.sync_copy(stokbuf.at[src], stok_hbm.at[dst])
            pltpu.sync_copy(gidxbuf.at[src], gidx_hbm.at[dst])
            return 0

        lax.fori_loop(0, ntail, tail, 0)

        @pl.when(sid == 0)
        def _():
            for i in range(8):
                texbuf[pl.ds(pl.multiple_of(i * 16, 16), 16)] = (
                    jnp.zeros((16,), jnp.int32))
            for e in range(E):
                r = e // 16
                lane = e % 16
                m_l = iota16 == lane
                ts_s = jnp.sum(jnp.where(m_l, ts_vregs[r], 0))
                tc_s = jnp.sum(jnp.where(m_l, t_vregs[r], 0))
                ew_vec = jnp.full((16,), e, jnp.int32)
                for half in range(2):
                    loc = iota16 + 16 * half
                    plsc.store_scatter(texbuf, [ts_s + loc], ew_vec,
                                       mask=loc < tc_s)
            pltpu.sync_copy(texbuf, texp_hbm)


def _route_sc(eflat, ewflat):
    f = pl.kernel(
        _route_body,
        out_type=[
            jax.ShapeDtypeStruct((PAD,), jnp.float32),
            jax.ShapeDtypeStruct((PAD,), jnp.int32),
            jax.ShapeDtypeStruct((PAD,), jnp.int32),
            jax.ShapeDtypeStruct((128,), jnp.int32),
        ],
        mesh=_SC_MESH,
        compiler_params=pltpu.CompilerParams(needs_layout_passes=False),
        scratch_types=[
            pltpu.VMEM((NPAIR,), jnp.int32),
            pltpu.VMEM((NPAIR,), jnp.float32),
            pltpu.VMEM((_EPW * NPAIR,), jnp.int32),
            pltpu.VMEM((NPAIR,), jnp.float32),
            pltpu.VMEM((NPAIR,), jnp.int32),
            pltpu.VMEM((NPAIR,), jnp.int32),
            pltpu.VMEM((64,), jnp.int32),
            pltpu.VMEM((128,), jnp.int32),
        ],
    )
    return f(eflat, ewflat)


def _gather_body(gidx_hbm, x_hbm, xs_hbm, idx_v, rowbuf, sem):
    cid = lax.axis_index("c")
    sid = lax.axis_index("s")
    w = sid * 2 + cid
    pltpu.sync_copy(gidx_hbm.at[w], idx_v)
    for j in range(_NCH):
        pltpu.async_copy(x_hbm.at[idx_v.at[j]], rowbuf, sem).wait()
        pltpu.sync_copy(rowbuf, xs_hbm.at[pl.ds(w * _WROWS + j * _RCH, _RCH)])


def _gather_sc(gidx, x2d):
    f = pl.kernel(
        _gather_body,
        out_type=jax.ShapeDtypeStruct((PAD, D), jnp.float32),
        mesh=_SC_MESH,
        compiler_params=pltpu.CompilerParams(needs_layout_passes=False),
        scratch_types=[
            pltpu.VMEM((_NCH, _RCH), jnp.int32),
            pltpu.VMEM((_RCH, D), jnp.float32),
            pltpu.SemaphoreType.DMA,
        ],
    )
    return f(gidx.reshape(32, _NCH, _RCH), x2d)


def _scatter_body(stok_hbm, yp_hbm, y01_hbm, idx_v, rowbuf, sem):
    cid = lax.axis_index("c")
    sid = lax.axis_index("s")
    w = sid * 2 + cid
    pltpu.sync_copy(stok_hbm.at[w], idx_v)
    for j in range(_NCH):
        pltpu.sync_copy(yp_hbm.at[pl.ds(w * _WROWS + j * _RCH, _RCH)], rowbuf)
        pltpu.async_copy(rowbuf, y01_hbm.at[idx_v.at[j]], sem).wait()


def _scatter_sc(stok, yp):
    f = pl.kernel(
        _scatter_body,
        out_type=jax.ShapeDtypeStruct((NPAIR + 8, D), jnp.float32),
        mesh=_SC_MESH,
        compiler_params=pltpu.CompilerParams(needs_layout_passes=False),
        scratch_types=[
            pltpu.VMEM((_NCH, _RCH), jnp.int32),
            pltpu.VMEM((_RCH, D), jnp.float32),
            pltpu.SemaphoreType.DMA,
        ],
    )
    return f(stok.reshape(32, _NCH, _RCH), yp)


def _ffn_body(texp_ref, xs_ref, w1_ref, b1_ref, w3_ref, b3_ref, w2_ref, b2_ref,
              pw_ref, yp_ref):
    xb = xs_ref[...]
    h1 = lax.dot_general(xb, w1_ref[0], (((1,), (1,)), ((), ())),
                         preferred_element_type=jnp.float32) + b1_ref[0]
    h3 = lax.dot_general(xb, w3_ref[0], (((1,), (1,)), ((), ())),
                         preferred_element_type=jnp.float32) + b3_ref[0]
    hp = h1 * h3
    h = hp * jax.nn.sigmoid(hp)
    o = lax.dot_general(h, w2_ref[0], (((1,), (1,)), ((), ())),
                        preferred_element_type=jnp.float32) + b2_ref[0]
    yp_ref[...] = o * pw_ref[...]


def _expert_ffn(texp, xs, W1, b1, W3, b3, W2, b2, pw2):
    b1r = b1.reshape(E, 1, F)
    b3r = b3.reshape(E, 1, F)
    b2r = b2.reshape(E, 1, D)
    grid_spec = pltpu.PrefetchScalarGridSpec(
        num_scalar_prefetch=1,
        grid=(G,),
        in_specs=[
            pl.BlockSpec((BT, D), lambda g, s: (g, 0)),
            pl.BlockSpec((1, F, D), lambda g, s: (s[g], 0, 0)),
            pl.BlockSpec((1, 1, F), lambda g, s: (s[g], 0, 0)),
            pl.BlockSpec((1, F, D), lambda g, s: (s[g], 0, 0)),
            pl.BlockSpec((1, 1, F), lambda g, s: (s[g], 0, 0)),
            pl.BlockSpec((1, D, F), lambda g, s: (s[g], 0, 0)),
            pl.BlockSpec((1, 1, D), lambda g, s: (s[g], 0, 0)),
            pl.BlockSpec((BT, 1), lambda g, s: (g, 0)),
        ],
        out_specs=pl.BlockSpec((BT, D), lambda g, s: (g, 0)),
    )
    return pl.pallas_call(
        _ffn_body,
        grid_spec=grid_spec,
        out_shape=jax.ShapeDtypeStruct((PAD, D), jnp.float32),
    )(texp, xs, W1, b1r, W3, b3r, W2, b2r, pw2)


def _shared_body(x_ref, ws1_ref, bs1_ref, ws3_ref, bs3_ref, ws2_ref, bs2_ref,
                 y01_ref, y_ref):
    xb = x_ref[...]
    h1 = lax.dot_general(xb, ws1_ref[...], (((1,), (1,)), ((), ())),
                         preferred_element_type=jnp.float32) + bs1_ref[...]
    h3 = lax.dot_general(xb, ws3_ref[...], (((1,), (1,)), ((), ())),
                         preferred_element_type=jnp.float32) + bs3_ref[...]
    hp = h1 * h3
    h = hp * jax.nn.sigmoid(hp)
    o = lax.dot_general(h, ws2_ref[...], (((1,), (1,)), ((), ())),
                        preferred_element_type=jnp.float32) + bs2_ref[...]
    yb = y01_ref[...]
    y_ref[...] = o + yb[:, :D] + yb[:, D:]


def _shared(x2d, Ws1, bs1, Ws3, bs3, Ws2, bs2, y01r):
    SB = 128
    grid = (T // SB,)
    return pl.pallas_call(
        _shared_body,
        grid=grid,
        in_specs=[
            pl.BlockSpec((SB, D), lambda i: (i, 0)),
            pl.BlockSpec((FS, D), lambda i: (0, 0)),
            pl.BlockSpec((1, FS), lambda i: (0, 0)),
            pl.BlockSpec((FS, D), lambda i: (0, 0)),
            pl.BlockSpec((1, FS), lambda i: (0, 0)),
            pl.BlockSpec((D, FS), lambda i: (0, 0)),
            pl.BlockSpec((1, D), lambda i: (0, 0)),
            pl.BlockSpec((SB, 2 * D), lambda i: (i, 0)),
        ],
        out_specs=pl.BlockSpec((SB, D), lambda i: (i, 0)),
        out_shape=jax.ShapeDtypeStruct((T, D), jnp.float32),
    )(x2d, Ws1, bs1.reshape(1, FS), Ws3, bs3.reshape(1, FS), Ws2,
      bs2.reshape(1, D), y01r)


def kernel(x, gate_w, W1, b1, W2, b2, W3, b3, Ws1, bs1, Ws2, bs2, Ws3, bs3):
    shape = x.shape
    x2d = x.reshape(T, D)

    ei0, ei1, ew0, ew1 = _gate(x2d, gate_w)
    eflat = jnp.concatenate([ei0, ei1], axis=1).reshape(-1)
    ewflat = jnp.concatenate([ew0, ew1], axis=1).reshape(-1)

    pw, stok, gidx, texp = _route_sc(eflat, ewflat)

    xs = _gather_sc(gidx, x2d)
    yp = _expert_ffn(texp[:G], xs, W1, b1, W3, b3, W2, b2, pw.reshape(PAD, 1))

    y01 = _scatter_sc(stok, yp)
    y01r = y01[:NPAIR].reshape(T, 2 * D)

    y = _shared(x2d, Ws1, bs1, Ws3, bs3, Ws2, bs2, y01r)
    return y.reshape(shape)


# speedup vs baseline: 1.0048x; 1.0048x over previous
import functools

import jax
import jax.numpy as jnp
from jax import lax
from jax.experimental import pallas as pl
from jax.experimental.pallas import tpu as pltpu
from jax.experimental.pallas import tpu_sc as plsc

D = 1024
F = 256
E = 64
T = 2048
FS = 2048
BT = 128
G = 96
PAD = G * BT
NPAIR = 2 * T
TRASH = NPAIR

_NEG = -1e30


def _gate_body(x_ref, gw_ref, ei0_ref, ei1_ref, ew0_ref, ew1_ref):
    xb = x_ref[...]
    l = lax.dot_general(xb, gw_ref[...], (((1,), (1,)), ((), ())),
                        preferred_element_type=jnp.float32)
    m0 = jnp.max(l, axis=1, keepdims=True)
    z = jnp.sum(jnp.exp(l - m0), axis=1, keepdims=True)
    ids = lax.broadcasted_iota(jnp.int32, l.shape, 1)
    a0 = jnp.min(jnp.where(l == m0, ids, E), axis=1, keepdims=True)
    lm = jnp.where(ids == a0, _NEG, l)
    m1 = jnp.max(lm, axis=1, keepdims=True)
    a1 = jnp.min(jnp.where(lm == m1, ids, E), axis=1, keepdims=True)
    p0 = 1.0 / z
    p1 = jnp.exp(m1 - m0) / z
    e0 = jnp.exp(p0)
    e1 = jnp.exp(p1)
    s = e0 + e1
    ei0_ref[...] = a0
    ei1_ref[...] = a1
    ew0_ref[...] = e0 / s
    ew1_ref[...] = e1 / s


def _gate(x2d, gate_w):
    TB = 256
    grid = (T // TB,)
    out = pl.pallas_call(
        _gate_body,
        grid=grid,
        in_specs=[
            pl.BlockSpec((TB, D), lambda i: (i, 0)),
            pl.BlockSpec((E, D), lambda i: (0, 0)),
        ],
        out_specs=[
            pl.BlockSpec((TB, 1), lambda i: (i, 0)),
            pl.BlockSpec((TB, 1), lambda i: (i, 0)),
            pl.BlockSpec((TB, 1), lambda i: (i, 0)),
            pl.BlockSpec((TB, 1), lambda i: (i, 0)),
        ],
        out_shape=[
            jax.ShapeDtypeStruct((T, 1), jnp.int32),
            jax.ShapeDtypeStruct((T, 1), jnp.int32),
            jax.ShapeDtypeStruct((T, 1), jnp.float32),
            jax.ShapeDtypeStruct((T, 1), jnp.float32),
        ],
    )(x2d, gate_w)
    return out


_SC_MESH = plsc.VectorSubcoreMesh(core_axis_name="c", subcore_axis_name="s")
_EPW = 4
_RCH = 32
_WROWS = PAD // 32
_NCH = _WROWS // _RCH


def _route_body(ei_hbm, ew_hbm, pw_hbm, stok_hbm, gidx_hbm, texp_hbm,
                ei_v, ew_v, bins_v, pwbuf, stokbuf, gidxbuf, hist_v,
                texbuf):
    cid = lax.axis_index("c")
    sid = lax.axis_index("s")
    iota16 = lax.iota(jnp.int32, 16)

    @pl.when(cid == 0)
    def _():
        pltpu.sync_copy(ei_hbm, ei_v)
        pltpu.sync_copy(ew_hbm, ew_v)

        cnts = []
        for j4 in range(_EPW):
            e = sid * _EPW + j4

            def body(j, cnt, j4=j4, e=e):
                v = ei_v[pl.ds(pl.multiple_of(j * 16, 16), 16)]
                m = v == e
                mi = m.astype(jnp.int32)
                cs = plsc.cumsum(mi)
                pos = cnt + cs - mi + j4 * NPAIR
                plsc.store_scatter(bins_v, [pos], iota16 + j * 16, mask=m)
                return cnt + jnp.sum(mi)

            cnts.append(lax.fori_loop(0, NPAIR // 16, body, jnp.int32(0)))

        for r in range(4):
            hist_v[pl.ds(pl.multiple_of(16 * r, 16), 16)] = (
                jnp.zeros((16,), jnp.int32))
        ones16 = jnp.full((16,), 1, jnp.int32)

        def hbody(j, _):
            v = ei_v[pl.ds(pl.multiple_of(j * 16, 16), 16)]
            plsc.addupdate_scatter(hist_v, [v], ones16)
            return 0

        lax.fori_loop(0, NPAIR // 16, hbody, 0)

        t_vregs, ts_vregs = [], []
        base = jnp.int32(0)
        for r in range(4):
            c_r = hist_v[pl.ds(pl.multiple_of(16 * r, 16), 16)]
            t_r = lax.shift_right_logical(c_r + (BT - 1), 7)
            incl = plsc.cumsum(t_r)
            ts_r = incl - t_r + base
            base = base + jnp.sum(t_r)
            t_vregs.append(t_r)
            ts_vregs.append(ts_r)
        total_tiles = base

        for j4 in range(_EPW):
            e = sid * _EPW + j4
            cnt = cnts[j4]
            ts_e = jnp.int32(0)
            for r in range(4):
                ts_e = ts_e + jnp.sum(
                    jnp.where(iota16 + 16 * r == e, ts_vregs[r], 0))
            tcnt_e = lax.shift_right_logical(cnt + (BT - 1), 7)
            rowoff = ts_e * BT

            def fill(j, _, j4=j4, cnt=cnt):
                lanes = iota16 + j * 16
                m = lanes < cnt
                p = bins_v[pl.ds(j4 * NPAIR + j * 16, 16)]
                pv = jnp.where(m, p, 0)
                wv = jnp.where(m, plsc.load_gather(ew_v, [pv]), 0.0)
                off = pl.ds(pl.multiple_of(j * 16, 16), 16)
                pwbuf[off] = wv
                stokbuf[off] = jnp.where(m, pv, TRASH)
                gidxbuf[off] = lax.shift_right_logical(pv, 1)
                return 0

            lax.fori_loop(0, tcnt_e * (BT // 16), fill, 0)

            def dump(t2, _, rowoff=rowoff):
                src = pl.ds(pl.multiple_of(t2 * BT, BT), BT)
                dst = pl.ds(pl.multiple_of(rowoff + t2 * BT, BT), BT)
                pltpu.sync_copy(pwbuf.at[src], pw_hbm.at[dst])
                pltpu.sync_copy(stokbuf.at[src], stok_hbm.at[dst])
                pltpu.sync_copy(gidxbuf.at[src], gidx_hbm.at[dst])
                return 0

            lax.fori_loop(0, tcnt_e, dump, 0)

        for i in range(BT // 16):
            off = pl.ds(pl.multiple_of(i * 16, 16), 16)
            pwbuf[off] = jnp.zeros((16,), jnp.float32)
            stokbuf[off] = jnp.full((16,), TRASH, jnp.int32)
            gidxbuf[off] = jnp.zeros((16,), jnp.int32)
        ntail = jnp.maximum(0, (G - total_tiles - sid + 15)) // 16

        def tail(k, _):
            g = total_tiles + sid + 16 * k
            dst = pl.ds(pl.multiple_of(g * BT, BT), BT)
            src = pl.ds(0, BT)
            pltpu.sync_copy(pwbuf.at[src], pw_hbm.at[dst])
            pltpu.sync_copy(stokbuf.at[src], stok_hbm.at[dst])
            pltpu.sync_copy(gidxbuf.at[src], gidx_hbm.at[dst])
            return 0

        lax.fori_loop(0, ntail, tail, 0)

        @pl.when(sid == 0)
        def _():
            for i in range(8):
                texbuf[pl.ds(pl.multiple_of(i * 16, 16), 16)] = (
                    jnp.zeros((16,), jnp.int32))
            for e in range(E):
                r = e // 16
                lane = e % 16
                m_l = iota16 == lane
                ts_s = jnp.sum(jnp.where(m_l, ts_vregs[r], 0))
                tc_s = jnp.sum(jnp.where(m_l, t_vregs[r], 0))
                ew_vec = jnp.full((16,), e, jnp.int32)
                for half in range(2):
                    loc = iota16 + 16 * half
                    plsc.store_scatter(texbuf, [ts_s + loc], ew_vec,
                                       mask=loc < tc_s)
            pltpu.sync_copy(texbuf, texp_hbm)


def _route_sc(eflat, ewflat):
    f = pl.kernel(
        _route_body,
        out_type=[
            jax.ShapeDtypeStruct((PAD,), jnp.float32),
            jax.ShapeDtypeStruct((PAD,), jnp.int32),
            jax.ShapeDtypeStruct((PAD,), jnp.int32),
            jax.ShapeDtypeStruct((128,), jnp.int32),
        ],
        mesh=_SC_MESH,
        compiler_params=pltpu.CompilerParams(needs_layout_passes=False),
        scratch_types=[
            pltpu.VMEM((NPAIR,), jnp.int32),
            pltpu.VMEM((NPAIR,), jnp.float32),
            pltpu.VMEM((_EPW * NPAIR,), jnp.int32),
            pltpu.VMEM((NPAIR,), jnp.float32),
            pltpu.VMEM((NPAIR,), jnp.int32),
            pltpu.VMEM((NPAIR,), jnp.int32),
            pltpu.VMEM((64,), jnp.int32),
            pltpu.VMEM((128,), jnp.int32),
        ],
    )
    return f(eflat, ewflat)


def _gather_body(gidx_hbm, x_hbm, xs_hbm, idx_v, rowbuf0, rowbuf1, sem0,
                 sem1):
    cid = lax.axis_index("c")
    sid = lax.axis_index("s")
    w = sid * 2 + cid
    bufs = (rowbuf0, rowbuf1)
    sems = (sem0, sem1)
    pltpu.sync_copy(gidx_hbm.at[w], idx_v)

    def start(j):
        return pltpu.async_copy(x_hbm.at[idx_v.at[j]], bufs[j % 2],
                                sems[j % 2])

    d = start(0)
    for j in range(_NCH):
        nxt = start(j + 1) if j + 1 < _NCH else None
        d.wait()
        pltpu.sync_copy(bufs[j % 2],
                        xs_hbm.at[pl.ds(w * _WROWS + j * _RCH, _RCH)])
        d = nxt


def _gather_sc(gidx, x2d):
    f = pl.kernel(
        _gather_body,
        out_type=jax.ShapeDtypeStruct((PAD, D), jnp.float32),
        mesh=_SC_MESH,
        compiler_params=pltpu.CompilerParams(needs_layout_passes=False),
        scratch_types=[
            pltpu.VMEM((_NCH, _RCH), jnp.int32),
            pltpu.VMEM((_RCH, D), jnp.float32),
            pltpu.VMEM((_RCH, D), jnp.float32),
            pltpu.SemaphoreType.DMA,
            pltpu.SemaphoreType.DMA,
        ],
    )
    return f(gidx.reshape(32, _NCH, _RCH), x2d)


def _scatter_body(stok_hbm, yp_hbm, y01_hbm, idx_v, rowbuf0, rowbuf1, sem0,
                  sem1):
    cid = lax.axis_index("c")
    sid = lax.axis_index("s")
    w = sid * 2 + cid
    bufs = (rowbuf0, rowbuf1)
    sems = (sem0, sem1)
    pltpu.sync_copy(stok_hbm.at[w], idx_v)

    def start(j):
        return pltpu.async_copy(
            yp_hbm.at[pl.ds(w * _WROWS + j * _RCH, _RCH)], bufs[j % 2],
            sems[j % 2])

    d = start(0)
    for j in range(_NCH):
        nxt = start(j + 1) if j + 1 < _NCH else None
        d.wait()
        pltpu.async_copy(bufs[j % 2], y01_hbm.at[idx_v.at[j]],
                         sems[j % 2]).wait()
        d = nxt


def _scatter_sc(stok, yp):
    f = pl.kernel(
        _scatter_body,
        out_type=jax.ShapeDtypeStruct((NPAIR + 8, D), jnp.float32),
        mesh=_SC_MESH,
        compiler_params=pltpu.CompilerParams(needs_layout_passes=False),
        scratch_types=[
            pltpu.VMEM((_NCH, _RCH), jnp.int32),
            pltpu.VMEM((_RCH, D), jnp.float32),
            pltpu.VMEM((_RCH, D), jnp.float32),
            pltpu.SemaphoreType.DMA,
            pltpu.SemaphoreType.DMA,
        ],
    )
    return f(stok.reshape(32, _NCH, _RCH), yp)


def _ffn_body(texp_ref, xs_ref, w1_ref, b1_ref, w3_ref, b3_ref, w2_ref, b2_ref,
              pw_ref, yp_ref):
    xb = xs_ref[...]
    h1 = lax.dot_general(xb, w1_ref[0], (((1,), (1,)), ((), ())),
                         preferred_element_type=jnp.float32) + b1_ref[0]
    h3 = lax.dot_general(xb, w3_ref[0], (((1,), (1,)), ((), ())),
                         preferred_element_type=jnp.float32) + b3_ref[0]
    hp = h1 * h3
    h = hp * jax.nn.sigmoid(hp)
    o = lax.dot_general(h, w2_ref[0], (((1,), (1,)), ((), ())),
                        preferred_element_type=jnp.float32) + b2_ref[0]
    yp_ref[...] = o * pw_ref[...]


def _expert_ffn(texp, xs, W1, b1, W3, b3, W2, b2, pw2):
    b1r = b1.reshape(E, 1, F)
    b3r = b3.reshape(E, 1, F)
    b2r = b2.reshape(E, 1, D)
    grid_spec = pltpu.PrefetchScalarGridSpec(
        num_scalar_prefetch=1,
        grid=(G,),
        in_specs=[
            pl.BlockSpec((BT, D), lambda g, s: (g, 0)),
            pl.BlockSpec((1, F, D), lambda g, s: (s[g], 0, 0)),
            pl.BlockSpec((1, 1, F), lambda g, s: (s[g], 0, 0)),
            pl.BlockSpec((1, F, D), lambda g, s: (s[g], 0, 0)),
            pl.BlockSpec((1, 1, F), lambda g, s: (s[g], 0, 0)),
            pl.BlockSpec((1, D, F), lambda g, s: (s[g], 0, 0)),
            pl.BlockSpec((1, 1, D), lambda g, s: (s[g], 0, 0)),
            pl.BlockSpec((BT, 1), lambda g, s: (g, 0)),
        ],
        out_specs=pl.BlockSpec((BT, D), lambda g, s: (g, 0)),
    )
    return pl.pallas_call(
        _ffn_body,
        grid_spec=grid_spec,
        out_shape=jax.ShapeDtypeStruct((PAD, D), jnp.float32),
    )(texp, xs, W1, b1r, W3, b3r, W2, b2r, pw2)


def _shared_body(x_ref, ws1_ref, bs1_ref, ws3_ref, bs3_ref, ws2_ref, bs2_ref,
                 y01_ref, y_ref):
    xb = x_ref[...]
    h1 = lax.dot_general(xb, ws1_ref[...], (((1,), (1,)), ((), ())),
                         preferred_element_type=jnp.float32) + bs1_ref[...]
    h3 = lax.dot_general(xb, ws3_ref[...], (((1,), (1,)), ((), ())),
                         preferred_element_type=jnp.float32) + bs3_ref[...]
    hp = h1 * h3
    h = hp * jax.nn.sigmoid(hp)
    o = lax.dot_general(h, ws2_ref[...], (((1,), (1,)), ((), ())),
                        preferred_element_type=jnp.float32) + bs2_ref[...]
    yb = y01_ref[...]
    y_ref[...] = o + yb[:, :D] + yb[:, D:]


def _shared(x2d, Ws1, bs1, Ws3, bs3, Ws2, bs2, y01r):
    SB = 128
    grid = (T // SB,)
    return pl.pallas_call(
        _shared_body,
        grid=grid,
        in_specs=[
            pl.BlockSpec((SB, D), lambda i: (i, 0)),
            pl.BlockSpec((FS, D), lambda i: (0, 0)),
            pl.BlockSpec((1, FS), lambda i: (0, 0)),
            pl.BlockSpec((FS, D), lambda i: (0, 0)),
            pl.BlockSpec((1, FS), lambda i: (0, 0)),
            pl.BlockSpec((D, FS), lambda i: (0, 0)),
            pl.BlockSpec((1, D), lambda i: (0, 0)),
            pl.BlockSpec((SB, 2 * D), lambda i: (i, 0)),
        ],
        out_specs=pl.BlockSpec((SB, D), lambda i: (i, 0)),
        out_shape=jax.ShapeDtypeStruct((T, D), jnp.float32),
    )(x2d, Ws1, bs1.reshape(1, FS), Ws3, bs3.reshape(1, FS), Ws2,
      bs2.reshape(1, D), y01r)


def kernel(x, gate_w, W1, b1, W2, b2, W3, b3, Ws1, bs1, Ws2, bs2, Ws3, bs3):
    shape = x.shape
    x2d = x.reshape(T, D)

    ei0, ei1, ew0, ew1 = _gate(x2d, gate_w)
    eflat = jnp.concatenate([ei0, ei1], axis=1).reshape(-1)
    ewflat = jnp.concatenate([ew0, ew1], axis=1).reshape(-1)

    pw, stok, gidx, texp = _route_sc(eflat, ewflat)

    xs = _gather_sc(gidx, x2d)
    yp = _expert_ffn(texp[:G], xs, W1, b1, W3, b3, W2, b2, pw.reshape(PAD, 1))

    y01 = _scatter_sc(stok, yp)
    y01r = y01[:NPAIR].reshape(T, 2 * D)

    y = _shared(x2d, Ws1, bs1, Ws3, bs3, Ws2, bs2, y01r)
    return y.reshape(shape)
